# baseline (speedup 1.0000x reference)
import functools

import jax
import jax.numpy as jnp
from jax import lax
from jax.experimental import pallas as pl
from jax.experimental.pallas import tpu as pltpu
from jax.experimental.pallas import tpu_sc as plsc

_N = 7056
_L = 16
_NS = 16
_NC = 2
_CHUNK = 448
_TAIL = _N - 15 * _CHUNK
_OUT_CHUNK = 224
_OUT_TAIL = _N - 31 * _OUT_CHUNK
_NEG = float("-inf")
_BIG = jnp.int32(2**31 - 1)


def _sc_body(x_hbm, out_hbm, in_v, stg_val, stg_idx, pub_val, pub_idx,
             loc_val, loc_idx, out_v):
    c = lax.axis_index("c")
    s = lax.axis_index("s")
    w = c * _NS + s

    @pl.when(s < _NS - 1)
    def _():
        pltpu.sync_copy(x_hbm.at[pl.ds(s * _CHUNK, _CHUNK)], in_v)

    @pl.when(s == _NS - 1)
    def _():
        pltpu.sync_copy(x_hbm.at[pl.ds((_NS - 1) * _CHUNK, _TAIL)],
                        in_v.at[pl.ds(0, _TAIL)])
        neg = jnp.full((_L,), _NEG, dtype=jnp.float32)
        for i in range(_TAIL // _L, _CHUNK // _L):
            in_v[pl.ds(i * _L, _L)] = neg

    lane = lax.iota(jnp.int32, _L)
    base = s * _CHUNK
    best_val = jnp.full((_L,), _NEG, dtype=jnp.float32)
    best_idx = jnp.zeros((_L,), dtype=jnp.int32)
    for i in range(_CHUNK // _L):
        v = in_v[pl.ds(i * _L, _L)]
        gidx = lane + (base + i * _L)
        take = v > best_val
        best_val = jnp.where(take, v, best_val)
        best_idx = jnp.where(take, gidx, best_idx)

    stg_val[...] = best_val
    stg_idx[...] = best_idx
    pltpu.sync_copy(stg_val, pub_val.at[s])
    pltpu.sync_copy(stg_idx, pub_idx.at[s])
    plsc.subcore_barrier()
    pltpu.sync_copy(pub_val, loc_val)
    pltpu.sync_copy(pub_idx, loc_idx)

    cur_val = jnp.full((_L,), _NEG, dtype=jnp.float32)
    cur_idx = jnp.full((_L,), _BIG, dtype=jnp.int32)
    for t in range(_NS):
        v = loc_val[t]
        i = loc_idx[t]
        take = (v > cur_val) | ((v == cur_val) & (i < cur_idx))
        cur_val = jnp.where(take, v, cur_val)
        cur_idx = jnp.where(take, i, cur_idx)

    m = jnp.max(cur_val)
    cand = jnp.where(cur_val == m, cur_idx, _BIG)
    winner = jnp.min(cand)

    out_base = w * _OUT_CHUNK
    for i in range(_OUT_CHUNK // _L):
        gidx = lane + (out_base + i * _L)
        out_v[pl.ds(i * _L, _L)] = (gidx == winner).astype(jnp.float32)

    @pl.when(w < _NC * _NS - 1)
    def _():
        pltpu.sync_copy(out_v, out_hbm.at[pl.ds(out_base, _OUT_CHUNK)])

    @pl.when(w == _NC * _NS - 1)
    def _():
        pltpu.sync_copy(out_v.at[pl.ds(0, _OUT_TAIL)],
                        out_hbm.at[pl.ds(out_base, _OUT_TAIL)])


def kernel(inputs):
    x = jnp.reshape(inputs, (_N,))
    sc_call = pl.kernel(
        _sc_body,
        out_type=jax.ShapeDtypeStruct((_N,), jnp.float32),
        mesh=plsc.VectorSubcoreMesh(core_axis_name="c", subcore_axis_name="s",
                                    num_cores=_NC, num_subcores=_NS),
        compiler_params=pltpu.CompilerParams(needs_layout_passes=False,
                                             use_tc_tiling_on_sc=False),
        scratch_types=[
            pltpu.VMEM((_CHUNK,), jnp.float32),
            pltpu.VMEM((_L,), jnp.float32),
            pltpu.VMEM((_L,), jnp.int32),
            pltpu.VMEM_SHARED((_NS, _L), jnp.float32),
            pltpu.VMEM_SHARED((_NS, _L), jnp.int32),
            pltpu.VMEM((_NS, _L), jnp.float32),
            pltpu.VMEM((_NS, _L), jnp.int32),
            pltpu.VMEM((_OUT_CHUNK,), jnp.float32),
        ],
    )
    return jnp.reshape(sc_call(x), (1, _N))


# speedup vs baseline: 1.0975x; 1.0975x over previous
import jax
import jax.numpy as jnp
from jax import lax
from jax.experimental import pallas as pl
from jax.experimental.pallas import tpu as pltpu
from jax.experimental.pallas import tpu_sc as plsc

_N = 7056
_L = 16
_NS = 16
_CHUNK = 448
_TAIL = _N - (_NS - 1) * _CHUNK
_NEG = float("-inf")
_BIG = jnp.int32(2**31 - 1)


def _sc_body(x_hbm, out_hbm, in_v, stg, pub, loc, out_v):
    s = lax.axis_index("s")

    @pl.when(s < _NS - 1)
    def _():
        pltpu.sync_copy(x_hbm.at[pl.ds(s * _CHUNK, _CHUNK)], in_v)

    @pl.when(s == _NS - 1)
    def _():
        pltpu.sync_copy(x_hbm.at[pl.ds((_NS - 1) * _CHUNK, _TAIL)],
                        in_v.at[pl.ds(0, _TAIL)])
        neg = jnp.full((_L,), _NEG, dtype=jnp.float32)
        for i in range(_TAIL // _L, _CHUNK // _L):
            in_v[pl.ds(i * _L, _L)] = neg

    lane = lax.iota(jnp.int32, _L)
    base = s * _CHUNK
    best_val = jnp.full((_L,), _NEG, dtype=jnp.float32)
    best_idx = jnp.zeros((_L,), dtype=jnp.int32)
    for i in range(_CHUNK // _L):
        v = in_v[pl.ds(i * _L, _L)]
        gidx = lane + (base + i * _L)
        take = v > best_val
        best_val = jnp.where(take, v, best_val)
        best_idx = jnp.where(take, gidx, best_idx)

    stg[0] = best_val
    stg[1] = plsc.bitcast(best_idx, jnp.float32)
    pltpu.sync_copy(stg, pub.at[s])
    plsc.subcore_barrier()
    pltpu.sync_copy(pub, loc)

    cur_val = jnp.full((_L,), _NEG, dtype=jnp.float32)
    cur_idx = jnp.full((_L,), _BIG, dtype=jnp.int32)
    for t in range(_NS):
        v = loc[t, 0]
        i = plsc.bitcast(loc[t, 1], jnp.int32)
        take = (v > cur_val) | ((v == cur_val) & (i < cur_idx))
        cur_val = jnp.where(take, v, cur_val)
        cur_idx = jnp.where(take, i, cur_idx)

    m = jnp.max(cur_val)
    cand = jnp.where(cur_val == m, cur_idx, _BIG)
    winner = jnp.min(cand)

    for i in range(_CHUNK // _L):
        gidx = lane + (base + i * _L)
        out_v[pl.ds(i * _L, _L)] = (gidx == winner).astype(jnp.float32)

    @pl.when(s < _NS - 1)
    def _():
        pltpu.sync_copy(out_v, out_hbm.at[pl.ds(base, _CHUNK)])

    @pl.when(s == _NS - 1)
    def _():
        pltpu.sync_copy(out_v.at[pl.ds(0, _TAIL)],
                        out_hbm.at[pl.ds(base, _TAIL)])


def kernel(inputs):
    x = jnp.reshape(inputs, (_N,))
    sc_call = pl.kernel(
        _sc_body,
        out_type=jax.ShapeDtypeStruct((_N,), jnp.float32),
        mesh=plsc.VectorSubcoreMesh(core_axis_name="c", subcore_axis_name="s",
                                    num_cores=1, num_subcores=_NS),
        compiler_params=pltpu.CompilerParams(needs_layout_passes=False,
                                             use_tc_tiling_on_sc=False,
                                             skip_device_barrier=True),
        scratch_types=[
            pltpu.VMEM((_CHUNK,), jnp.float32),
            pltpu.VMEM((2, _L), jnp.float32),
            pltpu.VMEM_SHARED((_NS, 2, _L), jnp.float32),
            pltpu.VMEM((_NS, 2, _L), jnp.float32),
            pltpu.VMEM((_CHUNK,), jnp.float32),
        ],
    )
    return jnp.reshape(sc_call(x), (1, _N))
